# Initial kernel scaffold; baseline (speedup 1.0000x reference)
import jax
import jax.numpy as jnp
from jax.experimental import pallas as pl


def kernel(predict, target):
    raise NotImplementedError("write your pallas kernel here")


# speedup vs baseline: 8.4383x; 8.4383x over previous
import jax
import jax.numpy as jnp
from jax.experimental import pallas as pl

THEA = 0.5
DELTA = 1.5
MIN_PIXELS = 20.0
R = 16
C = 32
NB = 4
N_PIX = 512 * 512
P = 8192
K = N_PIX // P


def _body(pred_ref, lab_ref, sums_ref, cnts_ref, rsq_ref, out_ref):
    ph = pl.program_id(0)
    b = pl.program_id(1)
    k = pl.program_id(2)
    feat = pred_ref[0]
    labs = lab_ref[0]
    oh = (labs == jax.lax.broadcasted_iota(jnp.int32, (R, P), 0)).astype(
        jnp.float32)

    @pl.when(ph == 0)
    def _():
        psums = jax.lax.dot_general(
            feat, oh, (((1,), (1,)), ((), ())),
            preferred_element_type=jnp.float32,
            precision=jax.lax.Precision.HIGHEST)
        pcnt = jax.lax.dot_general(
            jnp.ones((1, P), jnp.float32), oh, (((1,), (1,)), ((), ())),
            preferred_element_type=jnp.float32,
            precision=jax.lax.Precision.HIGHEST)

        @pl.when(k == 0)
        def _():
            sums_ref[pl.ds(b, 1)] = psums[None]
            cnts_ref[pl.ds(b, 1)] = pcnt[None]

        @pl.when(k != 0)
        def _():
            sums_ref[pl.ds(b, 1)] += psums[None]
            cnts_ref[pl.ds(b, 1)] += pcnt[None]

    @pl.when(ph == 1)
    def _():
        cnt_b = cnts_ref[pl.ds(b, 1)][0]
        safe = jnp.maximum(cnt_b, 1.0)
        means_b = sums_ref[pl.ds(b, 1)][0] / safe
        meanpx = jax.lax.dot_general(
            means_b, oh, (((1,), (0,)), ((), ())),
            preferred_element_type=jnp.float32,
            precision=jax.lax.Precision.HIGHEST)
        diff = feat - meanpx
        dsq = jnp.sum(diff * diff, axis=0, keepdims=True)
        d = jnp.sqrt(dsq)
        r = jnp.maximum(d - THEA, 0.0)
        r2 = r * r
        prsq = jax.lax.dot_general(
            r2, oh, (((1,), (1,)), ((), ())),
            preferred_element_type=jnp.float32,
            precision=jax.lax.Precision.HIGHEST)

        @pl.when(k == 0)
        def _():
            rsq_ref[pl.ds(b, 1)] = prsq[None]

        @pl.when(k != 0)
        def _():
            rsq_ref[pl.ds(b, 1)] += prsq[None]

        @pl.when((b == NB - 1) & (k == K - 1))
        def _():
            total = jnp.float32(0.0)
            for bb in range(NB):
                cnts2 = cnts_ref[bb]
                sums2 = sums_ref[bb]
                rsq2 = rsq_ref[bb]
                valid = (cnts2 > MIN_PIXELS).astype(jnp.float32)
                safe_c = jnp.maximum(cnts2, 1.0)
                means = sums2 / safe_c
                n_valid = jnp.maximum(jnp.sum(valid), 1.0)
                loss_var = jnp.sum(valid * (rsq2 / safe_c)) / n_valid
                diffp = means[:, :, None] - means[:, None, :]
                psq = jnp.sum(diffp * diffp, axis=0)
                pdist = jnp.sqrt(psq + 1e-12)
                eye = (jax.lax.broadcasted_iota(jnp.int32, (R, R), 0) ==
                       jax.lax.broadcasted_iota(jnp.int32, (R, R), 1))
                pm = (valid * valid[0][:, None]) * (1.0 - eye.astype(jnp.float32))
                rdis = jnp.maximum(2.0 * DELTA - pdist, 0.0)
                cntp = jnp.maximum(jnp.sum(pm), 1.0)
                loss_dis = jnp.sum(pm * rdis * rdis) / cntp
                mnorm = jnp.sqrt(jnp.sum(means * means, axis=0,
                                         keepdims=True))
                loss_reg = jnp.sum(valid * mnorm) / n_valid
                total = total + (loss_var + loss_dis + 0.001 * loss_reg)
            out_ref[...] = jnp.broadcast_to(total / NB, (1, 1))


def kernel(predict, target):
    pred = predict.reshape(NB, C, N_PIX)
    labs = target.reshape(NB * K, 1, P)
    outs = pl.pallas_call(
        _body,
        grid=(2, NB, K),
        in_specs=[
            pl.BlockSpec((1, C, P), lambda ph, b, k: (b, 0, k)),
            pl.BlockSpec((1, 1, P), lambda ph, b, k: (b * K + k, 0, 0)),
        ],
        out_specs=[
            pl.BlockSpec((NB, C, R), lambda ph, b, k: (0, 0, 0)),
            pl.BlockSpec((NB, 1, R), lambda ph, b, k: (0, 0, 0)),
            pl.BlockSpec((NB, 1, R), lambda ph, b, k: (0, 0, 0)),
            pl.BlockSpec((1, 1), lambda ph, b, k: (0, 0)),
        ],
        out_shape=[
            jax.ShapeDtypeStruct((NB, C, R), jnp.float32),
            jax.ShapeDtypeStruct((NB, 1, R), jnp.float32),
            jax.ShapeDtypeStruct((NB, 1, R), jnp.float32),
            jax.ShapeDtypeStruct((1, 1), jnp.float32),
        ],
    )(pred, labs)
    return outs[3][0, 0]


# speedup vs baseline: 12.6214x; 1.4957x over previous
import jax
import jax.numpy as jnp
from jax.experimental import pallas as pl

THEA = 0.5
DELTA = 1.5
MIN_PIXELS = 20.0
R = 16
C = 32
NB = 4
N_PIX = 512 * 512
P = 8192
K = N_PIX // P


def _body(pred_ref, lab_ref, sums_ref, cnts_ref, rsq_ref, out_ref):
    ph = pl.program_id(0)
    b = pl.program_id(1)
    k = pl.program_id(2)
    feat = pred_ref[0]
    labs = lab_ref[0].astype(jnp.bfloat16)
    iota = jax.lax.broadcasted_iota(jnp.int32, (R, P), 0).astype(jnp.bfloat16)
    oh = jnp.where(labs == iota, jnp.bfloat16(1), jnp.bfloat16(0))
    featb = feat.astype(jnp.bfloat16)

    @pl.when(ph == 0)
    def _():
        psums = jax.lax.dot_general(
            featb, oh, (((1,), (1,)), ((), ())),
            preferred_element_type=jnp.float32)
        pcnt = jax.lax.dot_general(
            jnp.ones((1, P), jnp.bfloat16), oh, (((1,), (1,)), ((), ())),
            preferred_element_type=jnp.float32)

        @pl.when(k == 0)
        def _():
            sums_ref[pl.ds(b, 1)] = psums[None]
            cnts_ref[pl.ds(b, 1)] = pcnt[None]

        @pl.when(k != 0)
        def _():
            sums_ref[pl.ds(b, 1)] += psums[None]
            cnts_ref[pl.ds(b, 1)] += pcnt[None]

    @pl.when(ph == 1)
    def _():
        cnt_b = cnts_ref[pl.ds(b, 1)][0]
        safe = jnp.maximum(cnt_b, 1.0)
        means_b = sums_ref[pl.ds(b, 1)][0] / safe
        meanpx = jax.lax.dot_general(
            means_b.astype(jnp.bfloat16), oh, (((1,), (0,)), ((), ())),
            preferred_element_type=jnp.float32)
        diff = feat - meanpx
        dsq = jax.lax.dot_general(
            jnp.ones((1, C), jnp.bfloat16), (diff * diff).astype(jnp.bfloat16),
            (((1,), (0,)), ((), ())),
            preferred_element_type=jnp.float32)
        d = jnp.sqrt(dsq)
        r = jnp.maximum(d - THEA, 0.0)
        r2 = (r * r).astype(jnp.bfloat16)
        prsq = jax.lax.dot_general(
            r2, oh, (((1,), (1,)), ((), ())),
            preferred_element_type=jnp.float32)

        @pl.when(k == 0)
        def _():
            rsq_ref[pl.ds(b, 1)] = prsq[None]

        @pl.when(k != 0)
        def _():
            rsq_ref[pl.ds(b, 1)] += prsq[None]

        @pl.when((b == NB - 1) & (k == K - 1))
        def _():
            total = jnp.float32(0.0)
            for bb in range(NB):
                cnts2 = cnts_ref[bb]
                sums2 = sums_ref[bb]
                rsq2 = rsq_ref[bb]
                valid = (cnts2 > MIN_PIXELS).astype(jnp.float32)
                safe_c = jnp.maximum(cnts2, 1.0)
                means = sums2 / safe_c
                n_valid = jnp.maximum(jnp.sum(valid), 1.0)
                loss_var = jnp.sum(valid * (rsq2 / safe_c)) / n_valid
                diffp = means[:, :, None] - means[:, None, :]
                psq = jnp.sum(diffp * diffp, axis=0)
                pdist = jnp.sqrt(psq + 1e-12)
                eye = (jax.lax.broadcasted_iota(jnp.int32, (R, R), 0) ==
                       jax.lax.broadcasted_iota(jnp.int32, (R, R), 1))
                pm = (valid * valid[0][:, None]) * (1.0 - eye.astype(jnp.float32))
                rdis = jnp.maximum(2.0 * DELTA - pdist, 0.0)
                cntp = jnp.maximum(jnp.sum(pm), 1.0)
                loss_dis = jnp.sum(pm * rdis * rdis) / cntp
                mnorm = jnp.sqrt(jnp.sum(means * means, axis=0,
                                         keepdims=True))
                loss_reg = jnp.sum(valid * mnorm) / n_valid
                total = total + (loss_var + loss_dis + 0.001 * loss_reg)
            out_ref[...] = jnp.broadcast_to(total / NB, (1, 1))


def kernel(predict, target):
    pred = predict.reshape(NB, C, N_PIX)
    labs = target.reshape(NB * K, 1, P)
    outs = pl.pallas_call(
        _body,
        grid=(2, NB, K),
        in_specs=[
            pl.BlockSpec((1, C, P), lambda ph, b, k: (b, 0, k)),
            pl.BlockSpec((1, 1, P), lambda ph, b, k: (b * K + k, 0, 0)),
        ],
        out_specs=[
            pl.BlockSpec((NB, C, R), lambda ph, b, k: (0, 0, 0)),
            pl.BlockSpec((NB, 1, R), lambda ph, b, k: (0, 0, 0)),
            pl.BlockSpec((NB, 1, R), lambda ph, b, k: (0, 0, 0)),
            pl.BlockSpec((1, 1), lambda ph, b, k: (0, 0)),
        ],
        out_shape=[
            jax.ShapeDtypeStruct((NB, C, R), jnp.float32),
            jax.ShapeDtypeStruct((NB, 1, R), jnp.float32),
            jax.ShapeDtypeStruct((NB, 1, R), jnp.float32),
            jax.ShapeDtypeStruct((1, 1), jnp.float32),
        ],
    )(pred, labs)
    return outs[3][0, 0]


# speedup vs baseline: 12.9724x; 1.0278x over previous
import jax
import jax.numpy as jnp
from jax.experimental import pallas as pl
from jax.experimental.pallas import tpu as pltpu

THEA = 0.5
DELTA = 1.5
MIN_PIXELS = 20.0
R = 16
C = 32
NB = 4
N_PIX = 512 * 512
P = 8192
K = N_PIX // P


def _onehot(lab_ref):
    labs = lab_ref[0]
    iota = jax.lax.broadcasted_iota(jnp.int32, (R, P), 0)
    return jnp.where(labs == iota, jnp.float32(1), jnp.float32(0))


def _pass1_body(pred_ref, lab_ref, sums_ref, cnts_ref):
    k = pl.program_id(1)
    feat = pred_ref[0]
    oh = _onehot(lab_ref)
    psums = jax.lax.dot_general(
        feat, oh, (((1,), (1,)), ((), ())),
        preferred_element_type=jnp.float32)
    pcnt = jax.lax.dot_general(
        jnp.ones((1, P), jnp.float32), oh, (((1,), (1,)), ((), ())),
        preferred_element_type=jnp.float32)

    @pl.when(k == 0)
    def _():
        sums_ref[...] = psums[None]
        cnts_ref[...] = pcnt[None]

    @pl.when(k != 0)
    def _():
        sums_ref[...] += psums[None]
        cnts_ref[...] += pcnt[None]


def _pass2_body(pred_ref, lab_ref, sums_ref, cnts_ref, rsq_ref, out_ref,
                means_ref):
    b = pl.program_id(0)
    k = pl.program_id(1)
    feat = pred_ref[0]
    oh = _onehot(lab_ref)

    @pl.when(k == 0)
    def _():
        cnt_b = cnts_ref[pl.ds(b, 1)][0]
        safe = jnp.maximum(cnt_b, 1.0)
        means_ref[...] = sums_ref[pl.ds(b, 1)][0] / safe

    meanpx = jax.lax.dot_general(
        means_ref[...], oh, (((1,), (0,)), ((), ())),
        preferred_element_type=jnp.float32)
    diff = feat - meanpx
    dsq = jax.lax.dot_general(
        jnp.ones((1, C), jnp.float32), diff * diff,
        (((1,), (0,)), ((), ())),
        preferred_element_type=jnp.float32)
    d = jnp.sqrt(dsq)
    r = jnp.maximum(d - THEA, 0.0)
    r2 = r * r
    prsq = jax.lax.dot_general(
        r2, oh, (((1,), (1,)), ((), ())),
        preferred_element_type=jnp.float32)

    @pl.when(k == 0)
    def _():
        rsq_ref[pl.ds(b, 1)] = prsq[None]

    @pl.when(k != 0)
    def _():
        rsq_ref[pl.ds(b, 1)] += prsq[None]

    @pl.when((b == NB - 1) & (k == K - 1))
    def _():
        total = jnp.float32(0.0)
        for bb in range(NB):
            cnts2 = cnts_ref[bb]
            sums2 = sums_ref[bb]
            rsq2 = rsq_ref[bb]
            valid = (cnts2 > MIN_PIXELS).astype(jnp.float32)
            safe_c = jnp.maximum(cnts2, 1.0)
            means = sums2 / safe_c
            n_valid = jnp.maximum(jnp.sum(valid), 1.0)
            loss_var = jnp.sum(valid * (rsq2 / safe_c)) / n_valid
            diffp = means[:, :, None] - means[:, None, :]
            psq = jnp.sum(diffp * diffp, axis=0)
            pdist = jnp.sqrt(psq + 1e-12)
            eye = (jax.lax.broadcasted_iota(jnp.int32, (R, R), 0) ==
                   jax.lax.broadcasted_iota(jnp.int32, (R, R), 1))
            pm = (valid * valid[0][:, None]) * (1.0 - eye.astype(jnp.float32))
            rdis = jnp.maximum(2.0 * DELTA - pdist, 0.0)
            cntp = jnp.maximum(jnp.sum(pm), 1.0)
            loss_dis = jnp.sum(pm * rdis * rdis) / cntp
            mnorm = jnp.sqrt(jnp.sum(means * means, axis=0,
                                     keepdims=True))
            loss_reg = jnp.sum(valid * mnorm) / n_valid
            total = total + (loss_var + loss_dis + 0.001 * loss_reg)
        out_ref[...] = jnp.broadcast_to(total / NB, (1, 1))


def kernel(predict, target):
    pred = predict.reshape(NB, C, N_PIX)
    labs = target.reshape(NB * K, 1, P)
    sums, cnts = pl.pallas_call(
        _pass1_body,
        grid=(NB, K),
        in_specs=[
            pl.BlockSpec((1, C, P), lambda b, k: (b, 0, k)),
            pl.BlockSpec((1, 1, P), lambda b, k: (b * K + k, 0, 0)),
        ],
        out_specs=[
            pl.BlockSpec((1, C, R), lambda b, k: (b, 0, 0)),
            pl.BlockSpec((1, 1, R), lambda b, k: (b, 0, 0)),
        ],
        out_shape=[
            jax.ShapeDtypeStruct((NB, C, R), jnp.float32),
            jax.ShapeDtypeStruct((NB, 1, R), jnp.float32),
        ],
    )(pred, labs)
    outs = pl.pallas_call(
        _pass2_body,
        grid=(NB, K),
        in_specs=[
            pl.BlockSpec((1, C, P), lambda b, k: (b, 0, k)),
            pl.BlockSpec((1, 1, P), lambda b, k: (b * K + k, 0, 0)),
            pl.BlockSpec((NB, C, R), lambda b, k: (0, 0, 0)),
            pl.BlockSpec((NB, 1, R), lambda b, k: (0, 0, 0)),
        ],
        out_specs=[
            pl.BlockSpec((NB, 1, R), lambda b, k: (0, 0, 0)),
            pl.BlockSpec((1, 1), lambda b, k: (0, 0)),
        ],
        out_shape=[
            jax.ShapeDtypeStruct((NB, 1, R), jnp.float32),
            jax.ShapeDtypeStruct((1, 1), jnp.float32),
        ],
        scratch_shapes=[pltpu.VMEM((C, R), jnp.float32)],
    )(pred, labs, sums, cnts)
    return outs[1][0, 0]


# speedup vs baseline: 15.5692x; 1.2002x over previous
import jax
import jax.numpy as jnp
from jax.experimental import pallas as pl
from jax.experimental.pallas import tpu as pltpu

THEA = 0.5
DELTA = 1.5
MIN_PIXELS = 20.0
R = 16
C = 32
NB = 4
N_PIX = 512 * 512
P = 16384
K = N_PIX // P


def _onehot(lab_ref):
    labs = lab_ref[0]
    iota = jax.lax.broadcasted_iota(jnp.int32, (R, P), 0)
    return jnp.where(labs == iota, jnp.float32(1), jnp.float32(0))


def _pass1_body(pred_ref, lab_ref, sums_ref, cnts_ref):
    k = pl.program_id(1)
    feat = pred_ref[0]
    oh = _onehot(lab_ref)
    psums = jax.lax.dot_general(
        feat, oh, (((1,), (1,)), ((), ())),
        preferred_element_type=jnp.float32)
    pcnt = jax.lax.dot_general(
        jnp.ones((1, P), jnp.float32), oh, (((1,), (1,)), ((), ())),
        preferred_element_type=jnp.float32)

    @pl.when(k == 0)
    def _():
        sums_ref[...] = psums[None]
        cnts_ref[...] = pcnt[None]

    @pl.when(k != 0)
    def _():
        sums_ref[...] += psums[None]
        cnts_ref[...] += pcnt[None]


def _pass2_body(pred_ref, lab_ref, sums_ref, cnts_ref, rsq_ref, out_ref,
                means_ref):
    b = pl.program_id(0)
    k = pl.program_id(1)
    feat = pred_ref[0]
    oh = _onehot(lab_ref)

    @pl.when(k == 0)
    def _():
        cnt_b = cnts_ref[pl.ds(b, 1)][0]
        safe = jnp.maximum(cnt_b, 1.0)
        means_ref[...] = sums_ref[pl.ds(b, 1)][0] / safe

    meanpx = jax.lax.dot_general(
        means_ref[...], oh, (((1,), (0,)), ((), ())),
        preferred_element_type=jnp.float32)
    diff = feat - meanpx
    dsq = jax.lax.dot_general(
        jnp.ones((1, C), jnp.float32), diff * diff,
        (((1,), (0,)), ((), ())),
        preferred_element_type=jnp.float32)
    d = jnp.sqrt(dsq)
    r = jnp.maximum(d - THEA, 0.0)
    r2 = r * r
    prsq = jax.lax.dot_general(
        r2, oh, (((1,), (1,)), ((), ())),
        preferred_element_type=jnp.float32)

    @pl.when(k == 0)
    def _():
        rsq_ref[pl.ds(b, 1)] = prsq[None]

    @pl.when(k != 0)
    def _():
        rsq_ref[pl.ds(b, 1)] += prsq[None]

    @pl.when((b == NB - 1) & (k == K - 1))
    def _():
        total = jnp.float32(0.0)
        for bb in range(NB):
            cnts2 = cnts_ref[bb]
            sums2 = sums_ref[bb]
            rsq2 = rsq_ref[bb]
            valid = (cnts2 > MIN_PIXELS).astype(jnp.float32)
            safe_c = jnp.maximum(cnts2, 1.0)
            means = sums2 / safe_c
            n_valid = jnp.maximum(jnp.sum(valid), 1.0)
            loss_var = jnp.sum(valid * (rsq2 / safe_c)) / n_valid
            diffp = means[:, :, None] - means[:, None, :]
            psq = jnp.sum(diffp * diffp, axis=0)
            pdist = jnp.sqrt(psq + 1e-12)
            eye = (jax.lax.broadcasted_iota(jnp.int32, (R, R), 0) ==
                   jax.lax.broadcasted_iota(jnp.int32, (R, R), 1))
            pm = (valid * valid[0][:, None]) * (1.0 - eye.astype(jnp.float32))
            rdis = jnp.maximum(2.0 * DELTA - pdist, 0.0)
            cntp = jnp.maximum(jnp.sum(pm), 1.0)
            loss_dis = jnp.sum(pm * rdis * rdis) / cntp
            mnorm = jnp.sqrt(jnp.sum(means * means, axis=0,
                                     keepdims=True))
            loss_reg = jnp.sum(valid * mnorm) / n_valid
            total = total + (loss_var + loss_dis + 0.001 * loss_reg)
        out_ref[...] = jnp.broadcast_to(total / NB, (1, 1))


def kernel(predict, target):
    pred = predict.reshape(NB, C, N_PIX)
    labs = target.reshape(NB * K, 1, P)
    sums, cnts = pl.pallas_call(
        _pass1_body,
        grid=(NB, K),
        in_specs=[
            pl.BlockSpec((1, C, P), lambda b, k: (b, 0, k)),
            pl.BlockSpec((1, 1, P), lambda b, k: (b * K + k, 0, 0)),
        ],
        out_specs=[
            pl.BlockSpec((1, C, R), lambda b, k: (b, 0, 0)),
            pl.BlockSpec((1, 1, R), lambda b, k: (b, 0, 0)),
        ],
        out_shape=[
            jax.ShapeDtypeStruct((NB, C, R), jnp.float32),
            jax.ShapeDtypeStruct((NB, 1, R), jnp.float32),
        ],
    )(pred, labs)
    outs = pl.pallas_call(
        _pass2_body,
        grid=(NB, K),
        in_specs=[
            pl.BlockSpec((1, C, P), lambda b, k: (b, 0, k)),
            pl.BlockSpec((1, 1, P), lambda b, k: (b * K + k, 0, 0)),
            pl.BlockSpec((NB, C, R), lambda b, k: (0, 0, 0)),
            pl.BlockSpec((NB, 1, R), lambda b, k: (0, 0, 0)),
        ],
        out_specs=[
            pl.BlockSpec((NB, 1, R), lambda b, k: (0, 0, 0)),
            pl.BlockSpec((1, 1), lambda b, k: (0, 0)),
        ],
        out_shape=[
            jax.ShapeDtypeStruct((NB, 1, R), jnp.float32),
            jax.ShapeDtypeStruct((1, 1), jnp.float32),
        ],
        scratch_shapes=[pltpu.VMEM((C, R), jnp.float32)],
    )(pred, labs, sums, cnts)
    return outs[1][0, 0]


# speedup vs baseline: 17.2902x; 1.1105x over previous
import jax
import jax.numpy as jnp
from jax.experimental import pallas as pl
from jax.experimental.pallas import tpu as pltpu

THEA = 0.5
DELTA = 1.5
MIN_PIXELS = 20.0
R = 16
C = 32
NB = 4
N_PIX = 512 * 512
P = 32768
K = N_PIX // P


def _onehot(lab_ref):
    labs = lab_ref[0]
    iota = jax.lax.broadcasted_iota(jnp.int32, (R, P), 0)
    return jnp.where(labs == iota, jnp.float32(1), jnp.float32(0))


def _pass1_body(pred_ref, lab_ref, sums_ref, cnts_ref):
    k = pl.program_id(1)
    feat = pred_ref[0]
    oh = _onehot(lab_ref)
    psums = jax.lax.dot_general(
        feat, oh, (((1,), (1,)), ((), ())),
        preferred_element_type=jnp.float32)
    pcnt = jax.lax.dot_general(
        jnp.ones((1, P), jnp.float32), oh, (((1,), (1,)), ((), ())),
        preferred_element_type=jnp.float32)

    @pl.when(k == 0)
    def _():
        sums_ref[...] = psums[None]
        cnts_ref[...] = pcnt[None]

    @pl.when(k != 0)
    def _():
        sums_ref[...] += psums[None]
        cnts_ref[...] += pcnt[None]


def _pass2_body(pred_ref, lab_ref, sums_ref, cnts_ref, rsq_ref, out_ref,
                means_ref):
    b = pl.program_id(0)
    k = pl.program_id(1)
    feat = pred_ref[0]
    oh = _onehot(lab_ref)

    @pl.when(k == 0)
    def _():
        cnt_b = cnts_ref[pl.ds(b, 1)][0]
        safe = jnp.maximum(cnt_b, 1.0)
        means_ref[...] = sums_ref[pl.ds(b, 1)][0] / safe

    meanpx = jax.lax.dot_general(
        means_ref[...], oh, (((1,), (0,)), ((), ())),
        preferred_element_type=jnp.float32)
    diff = feat - meanpx
    dsq = jax.lax.dot_general(
        jnp.ones((1, C), jnp.float32), diff * diff,
        (((1,), (0,)), ((), ())),
        preferred_element_type=jnp.float32)
    d = jnp.sqrt(dsq)
    r = jnp.maximum(d - THEA, 0.0)
    r2 = r * r
    prsq = jax.lax.dot_general(
        r2, oh, (((1,), (1,)), ((), ())),
        preferred_element_type=jnp.float32)

    @pl.when(k == 0)
    def _():
        rsq_ref[pl.ds(b, 1)] = prsq[None]

    @pl.when(k != 0)
    def _():
        rsq_ref[pl.ds(b, 1)] += prsq[None]

    @pl.when((b == NB - 1) & (k == K - 1))
    def _():
        total = jnp.float32(0.0)
        for bb in range(NB):
            cnts2 = cnts_ref[bb]
            sums2 = sums_ref[bb]
            rsq2 = rsq_ref[bb]
            valid = (cnts2 > MIN_PIXELS).astype(jnp.float32)
            safe_c = jnp.maximum(cnts2, 1.0)
            means = sums2 / safe_c
            n_valid = jnp.maximum(jnp.sum(valid), 1.0)
            loss_var = jnp.sum(valid * (rsq2 / safe_c)) / n_valid
            diffp = means[:, :, None] - means[:, None, :]
            psq = jnp.sum(diffp * diffp, axis=0)
            pdist = jnp.sqrt(psq + 1e-12)
            eye = (jax.lax.broadcasted_iota(jnp.int32, (R, R), 0) ==
                   jax.lax.broadcasted_iota(jnp.int32, (R, R), 1))
            pm = (valid * valid[0][:, None]) * (1.0 - eye.astype(jnp.float32))
            rdis = jnp.maximum(2.0 * DELTA - pdist, 0.0)
            cntp = jnp.maximum(jnp.sum(pm), 1.0)
            loss_dis = jnp.sum(pm * rdis * rdis) / cntp
            mnorm = jnp.sqrt(jnp.sum(means * means, axis=0,
                                     keepdims=True))
            loss_reg = jnp.sum(valid * mnorm) / n_valid
            total = total + (loss_var + loss_dis + 0.001 * loss_reg)
        out_ref[...] = jnp.broadcast_to(total / NB, (1, 1))


def kernel(predict, target):
    pred = predict.reshape(NB, C, N_PIX)
    labs = target.reshape(NB * K, 1, P)
    sums, cnts = pl.pallas_call(
        _pass1_body,
        grid=(NB, K),
        in_specs=[
            pl.BlockSpec((1, C, P), lambda b, k: (b, 0, k)),
            pl.BlockSpec((1, 1, P), lambda b, k: (b * K + k, 0, 0)),
        ],
        out_specs=[
            pl.BlockSpec((1, C, R), lambda b, k: (b, 0, 0)),
            pl.BlockSpec((1, 1, R), lambda b, k: (b, 0, 0)),
        ],
        out_shape=[
            jax.ShapeDtypeStruct((NB, C, R), jnp.float32),
            jax.ShapeDtypeStruct((NB, 1, R), jnp.float32),
        ],
    )(pred, labs)
    outs = pl.pallas_call(
        _pass2_body,
        grid=(NB, K),
        in_specs=[
            pl.BlockSpec((1, C, P), lambda b, k: (b, 0, k)),
            pl.BlockSpec((1, 1, P), lambda b, k: (b * K + k, 0, 0)),
            pl.BlockSpec((NB, C, R), lambda b, k: (0, 0, 0)),
            pl.BlockSpec((NB, 1, R), lambda b, k: (0, 0, 0)),
        ],
        out_specs=[
            pl.BlockSpec((NB, 1, R), lambda b, k: (0, 0, 0)),
            pl.BlockSpec((1, 1), lambda b, k: (0, 0)),
        ],
        out_shape=[
            jax.ShapeDtypeStruct((NB, 1, R), jnp.float32),
            jax.ShapeDtypeStruct((1, 1), jnp.float32),
        ],
        scratch_shapes=[pltpu.VMEM((C, R), jnp.float32)],
    )(pred, labs, sums, cnts)
    return outs[1][0, 0]


# speedup vs baseline: 18.0125x; 1.0418x over previous
import jax
import jax.numpy as jnp
from jax.experimental import pallas as pl
from jax.experimental.pallas import tpu as pltpu

THEA = 0.5
DELTA = 1.5
MIN_PIXELS = 20.0
R = 16
C = 32
NB = 4
N_PIX = 512 * 512
P = 65536
K = N_PIX // P


def _onehot(lab_ref):
    labs = lab_ref[0]
    iota = jax.lax.broadcasted_iota(jnp.int32, (R, P), 0)
    return jnp.where(labs == iota, jnp.float32(1), jnp.float32(0))


def _pass1_body(pred_ref, lab_ref, sums_ref, cnts_ref):
    k = pl.program_id(1)
    feat = pred_ref[0]
    oh = _onehot(lab_ref)
    psums = jax.lax.dot_general(
        feat, oh, (((1,), (1,)), ((), ())),
        preferred_element_type=jnp.float32)
    pcnt = jax.lax.dot_general(
        jnp.ones((1, P), jnp.float32), oh, (((1,), (1,)), ((), ())),
        preferred_element_type=jnp.float32)

    @pl.when(k == 0)
    def _():
        sums_ref[...] = psums[None]
        cnts_ref[...] = pcnt[None]

    @pl.when(k != 0)
    def _():
        sums_ref[...] += psums[None]
        cnts_ref[...] += pcnt[None]


def _pass2_body(pred_ref, lab_ref, sums_ref, cnts_ref, rsq_ref, out_ref,
                means_ref):
    b = pl.program_id(0)
    k = pl.program_id(1)
    feat = pred_ref[0]
    oh = _onehot(lab_ref)

    @pl.when(k == 0)
    def _():
        cnt_b = cnts_ref[pl.ds(b, 1)][0]
        safe = jnp.maximum(cnt_b, 1.0)
        means_ref[...] = sums_ref[pl.ds(b, 1)][0] / safe

    meanpx = jax.lax.dot_general(
        means_ref[...], oh, (((1,), (0,)), ((), ())),
        preferred_element_type=jnp.float32)
    diff = feat - meanpx
    dsq = jax.lax.dot_general(
        jnp.ones((1, C), jnp.float32), diff * diff,
        (((1,), (0,)), ((), ())),
        preferred_element_type=jnp.float32)
    d = jnp.sqrt(dsq)
    r = jnp.maximum(d - THEA, 0.0)
    r2 = r * r
    prsq = jax.lax.dot_general(
        r2, oh, (((1,), (1,)), ((), ())),
        preferred_element_type=jnp.float32)

    @pl.when(k == 0)
    def _():
        rsq_ref[pl.ds(b, 1)] = prsq[None]

    @pl.when(k != 0)
    def _():
        rsq_ref[pl.ds(b, 1)] += prsq[None]

    @pl.when((b == NB - 1) & (k == K - 1))
    def _():
        total = jnp.float32(0.0)
        for bb in range(NB):
            cnts2 = cnts_ref[bb]
            sums2 = sums_ref[bb]
            rsq2 = rsq_ref[bb]
            valid = (cnts2 > MIN_PIXELS).astype(jnp.float32)
            safe_c = jnp.maximum(cnts2, 1.0)
            means = sums2 / safe_c
            n_valid = jnp.maximum(jnp.sum(valid), 1.0)
            loss_var = jnp.sum(valid * (rsq2 / safe_c)) / n_valid
            diffp = means[:, :, None] - means[:, None, :]
            psq = jnp.sum(diffp * diffp, axis=0)
            pdist = jnp.sqrt(psq + 1e-12)
            eye = (jax.lax.broadcasted_iota(jnp.int32, (R, R), 0) ==
                   jax.lax.broadcasted_iota(jnp.int32, (R, R), 1))
            pm = (valid * valid[0][:, None]) * (1.0 - eye.astype(jnp.float32))
            rdis = jnp.maximum(2.0 * DELTA - pdist, 0.0)
            cntp = jnp.maximum(jnp.sum(pm), 1.0)
            loss_dis = jnp.sum(pm * rdis * rdis) / cntp
            mnorm = jnp.sqrt(jnp.sum(means * means, axis=0,
                                     keepdims=True))
            loss_reg = jnp.sum(valid * mnorm) / n_valid
            total = total + (loss_var + loss_dis + 0.001 * loss_reg)
        out_ref[...] = jnp.broadcast_to(total / NB, (1, 1))


def kernel(predict, target):
    pred = predict.reshape(NB, C, N_PIX)
    labs = target.reshape(NB * K, 1, P)
    sums, cnts = pl.pallas_call(
        _pass1_body,
        grid=(NB, K),
        in_specs=[
            pl.BlockSpec((1, C, P), lambda b, k: (b, 0, k)),
            pl.BlockSpec((1, 1, P), lambda b, k: (b * K + k, 0, 0)),
        ],
        out_specs=[
            pl.BlockSpec((1, C, R), lambda b, k: (b, 0, 0)),
            pl.BlockSpec((1, 1, R), lambda b, k: (b, 0, 0)),
        ],
        out_shape=[
            jax.ShapeDtypeStruct((NB, C, R), jnp.float32),
            jax.ShapeDtypeStruct((NB, 1, R), jnp.float32),
        ],
    )(pred, labs)
    outs = pl.pallas_call(
        _pass2_body,
        grid=(NB, K),
        in_specs=[
            pl.BlockSpec((1, C, P), lambda b, k: (b, 0, k)),
            pl.BlockSpec((1, 1, P), lambda b, k: (b * K + k, 0, 0)),
            pl.BlockSpec((NB, C, R), lambda b, k: (0, 0, 0)),
            pl.BlockSpec((NB, 1, R), lambda b, k: (0, 0, 0)),
        ],
        out_specs=[
            pl.BlockSpec((NB, 1, R), lambda b, k: (0, 0, 0)),
            pl.BlockSpec((1, 1), lambda b, k: (0, 0)),
        ],
        out_shape=[
            jax.ShapeDtypeStruct((NB, 1, R), jnp.float32),
            jax.ShapeDtypeStruct((1, 1), jnp.float32),
        ],
        scratch_shapes=[pltpu.VMEM((C, R), jnp.float32)],
    )(pred, labs, sums, cnts)
    return outs[1][0, 0]


# speedup vs baseline: 36.0564x; 2.0017x over previous
import jax
import jax.numpy as jnp
from jax.experimental import pallas as pl
from jax.experimental.pallas import tpu as pltpu

THEA = 0.5
DELTA = 1.5
MIN_PIXELS = 20.0
R = 16
C = 32
NB = 4
N_PIX = 512 * 512
P = 65536
K = N_PIX // P


def _onehot(lab_ref):
    labs = lab_ref[0]
    iota = jax.lax.broadcasted_iota(jnp.int32, (R, P), 0)
    return jnp.where(labs == iota, jnp.float32(1), jnp.float32(0))


def _pass1_body(pred_ref, lab_ref, sums_ref, cnts_ref):
    k = pl.program_id(1)
    feat = pred_ref[0].reshape(C, P)
    oh = _onehot(lab_ref)
    psums = jax.lax.dot_general(
        feat, oh, (((1,), (1,)), ((), ())),
        preferred_element_type=jnp.float32)
    pcnt = jax.lax.dot_general(
        jnp.ones((1, P), jnp.float32), oh, (((1,), (1,)), ((), ())),
        preferred_element_type=jnp.float32)

    @pl.when(k == 0)
    def _():
        sums_ref[...] = psums[None]
        cnts_ref[...] = pcnt[None]

    @pl.when(k != 0)
    def _():
        sums_ref[...] += psums[None]
        cnts_ref[...] += pcnt[None]


def _pass2_body(pred_ref, lab_ref, sums_ref, cnts_ref, rsq_ref, out_ref,
                means_ref):
    b = pl.program_id(0)
    k = pl.program_id(1)
    feat = pred_ref[0].reshape(C, P)
    oh = _onehot(lab_ref)

    @pl.when(k == 0)
    def _():
        cnt_b = cnts_ref[pl.ds(b, 1)][0]
        safe = jnp.maximum(cnt_b, 1.0)
        means_ref[...] = sums_ref[pl.ds(b, 1)][0] / safe

    meanpx = jax.lax.dot_general(
        means_ref[...], oh, (((1,), (0,)), ((), ())),
        preferred_element_type=jnp.float32)
    diff = feat - meanpx
    dsq = jax.lax.dot_general(
        jnp.ones((1, C), jnp.float32), diff * diff,
        (((1,), (0,)), ((), ())),
        preferred_element_type=jnp.float32)
    d = jnp.sqrt(dsq)
    r = jnp.maximum(d - THEA, 0.0)
    r2 = r * r
    prsq = jax.lax.dot_general(
        r2, oh, (((1,), (1,)), ((), ())),
        preferred_element_type=jnp.float32)

    @pl.when(k == 0)
    def _():
        rsq_ref[pl.ds(b, 1)] = prsq[None]

    @pl.when(k != 0)
    def _():
        rsq_ref[pl.ds(b, 1)] += prsq[None]

    @pl.when((b == NB - 1) & (k == K - 1))
    def _():
        total = jnp.float32(0.0)
        for bb in range(NB):
            cnts2 = cnts_ref[bb]
            sums2 = sums_ref[bb]
            rsq2 = rsq_ref[bb]
            valid = (cnts2 > MIN_PIXELS).astype(jnp.float32)
            safe_c = jnp.maximum(cnts2, 1.0)
            means = sums2 / safe_c
            n_valid = jnp.maximum(jnp.sum(valid), 1.0)
            loss_var = jnp.sum(valid * (rsq2 / safe_c)) / n_valid
            diffp = means[:, :, None] - means[:, None, :]
            psq = jnp.sum(diffp * diffp, axis=0)
            pdist = jnp.sqrt(psq + 1e-12)
            eye = (jax.lax.broadcasted_iota(jnp.int32, (R, R), 0) ==
                   jax.lax.broadcasted_iota(jnp.int32, (R, R), 1))
            pm = (valid * valid[0][:, None]) * (1.0 - eye.astype(jnp.float32))
            rdis = jnp.maximum(2.0 * DELTA - pdist, 0.0)
            cntp = jnp.maximum(jnp.sum(pm), 1.0)
            loss_dis = jnp.sum(pm * rdis * rdis) / cntp
            mnorm = jnp.sqrt(jnp.sum(means * means, axis=0,
                                     keepdims=True))
            loss_reg = jnp.sum(valid * mnorm) / n_valid
            total = total + (loss_var + loss_dis + 0.001 * loss_reg)
        out_ref[...] = jnp.broadcast_to(total / NB, (1, 1))


HR = P // 512


def kernel(predict, target):
    pred = predict
    labs = target.reshape(NB * K, 1, P)
    sums, cnts = pl.pallas_call(
        _pass1_body,
        grid=(NB, K),
        in_specs=[
            pl.BlockSpec((1, C, HR, 512), lambda b, k: (b, 0, k, 0)),
            pl.BlockSpec((1, 1, P), lambda b, k: (b * K + k, 0, 0)),
        ],
        out_specs=[
            pl.BlockSpec((1, C, R), lambda b, k: (b, 0, 0)),
            pl.BlockSpec((1, 1, R), lambda b, k: (b, 0, 0)),
        ],
        out_shape=[
            jax.ShapeDtypeStruct((NB, C, R), jnp.float32),
            jax.ShapeDtypeStruct((NB, 1, R), jnp.float32),
        ],
    )(pred, labs)
    outs = pl.pallas_call(
        _pass2_body,
        grid=(NB, K),
        in_specs=[
            pl.BlockSpec((1, C, HR, 512), lambda b, k: (b, 0, k, 0)),
            pl.BlockSpec((1, 1, P), lambda b, k: (b * K + k, 0, 0)),
            pl.BlockSpec((NB, C, R), lambda b, k: (0, 0, 0)),
            pl.BlockSpec((NB, 1, R), lambda b, k: (0, 0, 0)),
        ],
        out_specs=[
            pl.BlockSpec((NB, 1, R), lambda b, k: (0, 0, 0)),
            pl.BlockSpec((1, 1), lambda b, k: (0, 0)),
        ],
        out_shape=[
            jax.ShapeDtypeStruct((NB, 1, R), jnp.float32),
            jax.ShapeDtypeStruct((1, 1), jnp.float32),
        ],
        scratch_shapes=[pltpu.VMEM((C, R), jnp.float32)],
    )(pred, labs, sums, cnts)
    return outs[1][0, 0]


# speedup vs baseline: 37.1360x; 1.0299x over previous
import jax
import jax.numpy as jnp
from jax.experimental import pallas as pl
from jax.experimental.pallas import tpu as pltpu

THEA = 0.5
DELTA = 1.5
MIN_PIXELS = 20.0
R = 16
C = 32
NB = 4
N_PIX = 512 * 512
P = 131072
K = N_PIX // P


def _onehot(lab_ref):
    labs = lab_ref[0]
    iota = jax.lax.broadcasted_iota(jnp.int32, (R, P), 0)
    return jnp.where(labs == iota, jnp.float32(1), jnp.float32(0))


def _pass1_body(pred_ref, lab_ref, sums_ref, cnts_ref):
    k = pl.program_id(1)
    feat = pred_ref[0].reshape(C, P)
    oh = _onehot(lab_ref)
    psums = jax.lax.dot_general(
        feat, oh, (((1,), (1,)), ((), ())),
        preferred_element_type=jnp.float32)
    pcnt = jax.lax.dot_general(
        jnp.ones((1, P), jnp.float32), oh, (((1,), (1,)), ((), ())),
        preferred_element_type=jnp.float32)

    @pl.when(k == 0)
    def _():
        sums_ref[...] = psums[None]
        cnts_ref[...] = pcnt[None]

    @pl.when(k != 0)
    def _():
        sums_ref[...] += psums[None]
        cnts_ref[...] += pcnt[None]


def _pass2_body(pred_ref, lab_ref, sums_ref, cnts_ref, rsq_ref, out_ref,
                means_ref):
    b = pl.program_id(0)
    k = pl.program_id(1)
    feat = pred_ref[0].reshape(C, P)
    oh = _onehot(lab_ref)

    @pl.when(k == 0)
    def _():
        cnt_b = cnts_ref[pl.ds(b, 1)][0]
        safe = jnp.maximum(cnt_b, 1.0)
        means_ref[...] = sums_ref[pl.ds(b, 1)][0] / safe

    meanpx = jax.lax.dot_general(
        means_ref[...], oh, (((1,), (0,)), ((), ())),
        preferred_element_type=jnp.float32)
    diff = feat - meanpx
    dsq = jax.lax.dot_general(
        jnp.ones((1, C), jnp.float32), diff * diff,
        (((1,), (0,)), ((), ())),
        preferred_element_type=jnp.float32)
    d = jnp.sqrt(dsq)
    r = jnp.maximum(d - THEA, 0.0)
    r2 = r * r
    prsq = jax.lax.dot_general(
        r2, oh, (((1,), (1,)), ((), ())),
        preferred_element_type=jnp.float32)

    @pl.when(k == 0)
    def _():
        rsq_ref[pl.ds(b, 1)] = prsq[None]

    @pl.when(k != 0)
    def _():
        rsq_ref[pl.ds(b, 1)] += prsq[None]

    @pl.when((b == NB - 1) & (k == K - 1))
    def _():
        total = jnp.float32(0.0)
        for bb in range(NB):
            cnts2 = cnts_ref[bb]
            sums2 = sums_ref[bb]
            rsq2 = rsq_ref[bb]
            valid = (cnts2 > MIN_PIXELS).astype(jnp.float32)
            safe_c = jnp.maximum(cnts2, 1.0)
            means = sums2 / safe_c
            n_valid = jnp.maximum(jnp.sum(valid), 1.0)
            loss_var = jnp.sum(valid * (rsq2 / safe_c)) / n_valid
            diffp = means[:, :, None] - means[:, None, :]
            psq = jnp.sum(diffp * diffp, axis=0)
            pdist = jnp.sqrt(psq + 1e-12)
            eye = (jax.lax.broadcasted_iota(jnp.int32, (R, R), 0) ==
                   jax.lax.broadcasted_iota(jnp.int32, (R, R), 1))
            pm = (valid * valid[0][:, None]) * (1.0 - eye.astype(jnp.float32))
            rdis = jnp.maximum(2.0 * DELTA - pdist, 0.0)
            cntp = jnp.maximum(jnp.sum(pm), 1.0)
            loss_dis = jnp.sum(pm * rdis * rdis) / cntp
            mnorm = jnp.sqrt(jnp.sum(means * means, axis=0,
                                     keepdims=True))
            loss_reg = jnp.sum(valid * mnorm) / n_valid
            total = total + (loss_var + loss_dis + 0.001 * loss_reg)
        out_ref[...] = jnp.broadcast_to(total / NB, (1, 1))


HR = P // 512


def kernel(predict, target):
    pred = predict
    labs = target.reshape(NB * K, 1, P)
    sums, cnts = pl.pallas_call(
        _pass1_body,
        grid=(NB, K),
        in_specs=[
            pl.BlockSpec((1, C, HR, 512), lambda b, k: (b, 0, k, 0)),
            pl.BlockSpec((1, 1, P), lambda b, k: (b * K + k, 0, 0)),
        ],
        out_specs=[
            pl.BlockSpec((1, C, R), lambda b, k: (b, 0, 0)),
            pl.BlockSpec((1, 1, R), lambda b, k: (b, 0, 0)),
        ],
        out_shape=[
            jax.ShapeDtypeStruct((NB, C, R), jnp.float32),
            jax.ShapeDtypeStruct((NB, 1, R), jnp.float32),
        ],
    )(pred, labs)
    outs = pl.pallas_call(
        _pass2_body,
        grid=(NB, K),
        in_specs=[
            pl.BlockSpec((1, C, HR, 512), lambda b, k: (b, 0, k, 0)),
            pl.BlockSpec((1, 1, P), lambda b, k: (b * K + k, 0, 0)),
            pl.BlockSpec((NB, C, R), lambda b, k: (0, 0, 0)),
            pl.BlockSpec((NB, 1, R), lambda b, k: (0, 0, 0)),
        ],
        out_specs=[
            pl.BlockSpec((NB, 1, R), lambda b, k: (0, 0, 0)),
            pl.BlockSpec((1, 1), lambda b, k: (0, 0)),
        ],
        out_shape=[
            jax.ShapeDtypeStruct((NB, 1, R), jnp.float32),
            jax.ShapeDtypeStruct((1, 1), jnp.float32),
        ],
        scratch_shapes=[pltpu.VMEM((C, R), jnp.float32)],
    )(pred, labs, sums, cnts)
    return outs[1][0, 0]
